# baseline (speedup 1.0000x reference)
import functools

import jax
import jax.numpy as jnp
from jax import lax
from jax.experimental import pallas as pl
from jax.experimental.pallas import tpu as pltpu
from jax.experimental.pallas import tpu_sc as plsc

_LANES = 16
_IDX_CHUNK = 128


@functools.lru_cache(maxsize=None)
def _make_pool(V, E, B, CTX):
  info = plsc.get_sparse_core_info()
  nw = info.num_cores * info.num_subcores
  assert B % nw == 0
  b_per_w = B // nw
  n_idx = b_per_w * CTX
  assert n_idx % _IDX_CHUNK == 0
  n_chunks = n_idx // _IDX_CHUNK
  mesh = plsc.VectorSubcoreMesh(core_axis_name="c", subcore_axis_name="s")

  @functools.partial(
      pl.kernel,
      out_type=jax.ShapeDtypeStruct((B, E), jnp.float32),
      mesh=mesh,
      compiler_params=pltpu.CompilerParams(use_tc_tiling_on_sc=False),
      scratch_types=[
          pltpu.VMEM((n_idx,), jnp.int32),
          pltpu.VMEM((n_idx, E), jnp.float32),
          pltpu.VMEM((b_per_w, E), jnp.float32),
          pltpu.SemaphoreType.DMA,
      ],
  )
  def pool(table_hbm, idx_hbm, out_hbm, idx_v, rows_v, avg_v, sem):
    wid = lax.axis_index("s") * info.num_cores + lax.axis_index("c")
    base = wid * b_per_w
    pltpu.sync_copy(idx_hbm.at[pl.ds(base * CTX, n_idx)], idx_v)
    copies = []
    for j in range(n_chunks):
      copies.append(
          pltpu.async_copy(
              table_hbm.at[idx_v.at[pl.ds(j * _IDX_CHUNK, _IDX_CHUNK)]],
              rows_v.at[pl.ds(j * _IDX_CHUNK, _IDX_CHUNK)],
              sem,
          )
      )
    for c in copies:
      c.wait()
    scale = jnp.float32(1.0 / CTX)

    def row_body(r, carry):
      rbase = r * CTX
      for e in range(E // _LANES):
        sl = pl.ds(e * _LANES, _LANES)
        acc = rows_v[rbase, sl]
        for c in range(1, CTX):
          acc = acc + rows_v[rbase + c, sl]
        avg_v[r, sl] = acc * scale
      return carry

    lax.fori_loop(0, b_per_w, row_body, 0)
    pltpu.sync_copy(avg_v, out_hbm.at[pl.ds(base, b_per_w)])

  return pool


@functools.lru_cache(maxsize=None)
def _make_matmul_t2(B, E, V, BN=2048, NBUF=4):
  grid = pl.cdiv(V, BN)
  tail = V - (grid - 1) * BN
  Vp = grid * BN

  def mm(avg_ref, w_ref, b_ref, out_ref, scratch, sems):
    j = pl.program_id(0)
    slot = lax.rem(j, NBUF)

    @pl.when(j >= NBUF)
    def _wait_slot():
      pltpu.make_async_copy(
          scratch.at[slot], out_ref.at[pl.ds(0, BN), :], sems.at[slot]
      ).wait()

    prod = lax.dot_general(
        w_ref[...], avg_ref[...],
        dimension_numbers=(((0,), (1,)), ((), ())),
        preferred_element_type=jnp.float32,
    )
    scratch.at[slot][...] = prod + b_ref[...]

    @pl.when(j < grid - 1)
    def _copy_full():
      pltpu.make_async_copy(
          scratch.at[slot], out_ref.at[pl.ds(j * BN, BN), :], sems.at[slot]
      ).start()

    @pl.when(j == grid - 1)
    def _copy_tail_and_drain():
      pltpu.make_async_copy(
          scratch.at[slot].at[pl.ds(0, tail), :],
          out_ref.at[pl.ds((grid - 1) * BN, tail), :],
          sems.at[slot],
      ).start()
      for k in range(1, NBUF):
        s = lax.rem(j + k, NBUF)
        pltpu.make_async_copy(
            scratch.at[s], out_ref.at[pl.ds(0, BN), :], sems.at[s]
        ).wait()
      pltpu.make_async_copy(
          scratch.at[slot].at[pl.ds(0, tail), :],
          out_ref.at[pl.ds(0, tail), :],
          sems.at[slot],
      ).wait()

  return pl.pallas_call(
      mm,
      grid=(grid,),
      in_specs=[
          pl.BlockSpec((B, E), lambda j: (0, 0)),
          pl.BlockSpec((E, BN), lambda j: (0, j)),
          pl.BlockSpec((BN, 1), lambda j: (j, 0)),
      ],
      out_specs=pl.BlockSpec(memory_space=pl.ANY),
      out_shape=jax.ShapeDtypeStruct((V, B), jnp.float32),
      scratch_shapes=[
          pltpu.VMEM((NBUF, BN, B), jnp.float32),
          pltpu.SemaphoreType.DMA((NBUF,)),
      ],
  ), Vp


@functools.lru_cache(maxsize=None)
def _make_matmul_t(B, E, V, BN=4096):
  grid = pl.cdiv(V, BN)

  def mm(avg_ref, w_ref, b_ref, out_ref):
    prod = lax.dot_general(
        w_ref[...], avg_ref[...],
        dimension_numbers=(((0,), (1,)), ((), ())),
        preferred_element_type=jnp.float32,
    )
    out_ref[...] = prod + b_ref[...]

  return pl.pallas_call(
      mm,
      grid=(grid,),
      in_specs=[
          pl.BlockSpec((B, E), lambda j: (0, 0)),
          pl.BlockSpec((E, BN), lambda j: (0, j)),
          pl.BlockSpec((BN, 1), lambda j: (j, 0)),
      ],
      out_specs=pl.BlockSpec((BN, B), lambda j: (j, 0)),
      out_shape=jax.ShapeDtypeStruct((V, B), jnp.float32),
  )


def kernel(x, emb_table, W, b):
  B, CTX = x.shape
  V, E = emb_table.shape
  avg = _make_pool(V, E, B, CTX)(emb_table, x.reshape(B * CTX))
  call, Vp = _make_matmul_t2(B, E, V)
  Wp = jnp.pad(W, ((0, 0), (0, Vp - V)))
  bp = jnp.pad(b.reshape(V, 1), ((0, Vp - V), (0, 0)))
  out_t = call(avg, Wp, bp)
  return out_t.T


# speedup vs baseline: 1.1823x; 1.1823x over previous
import functools

import jax
import jax.numpy as jnp
from jax import lax
from jax.experimental import pallas as pl
from jax.experimental.pallas import tpu as pltpu
from jax.experimental.pallas import tpu_sc as plsc

_LANES = 16
_IDX_CHUNK = 128


@functools.lru_cache(maxsize=None)
def _make_pool(V, E, B, CTX):
  info = plsc.get_sparse_core_info()
  nw = info.num_cores * info.num_subcores
  assert B % nw == 0
  b_per_w = B // nw
  n_idx = b_per_w * CTX
  assert n_idx % _IDX_CHUNK == 0
  n_chunks = n_idx // _IDX_CHUNK
  mesh = plsc.VectorSubcoreMesh(core_axis_name="c", subcore_axis_name="s")

  @functools.partial(
      pl.kernel,
      out_type=jax.ShapeDtypeStruct((B, E), jnp.float32),
      mesh=mesh,
      compiler_params=pltpu.CompilerParams(use_tc_tiling_on_sc=False),
      scratch_types=[
          pltpu.VMEM((n_idx,), jnp.int32),
          pltpu.VMEM((n_idx, E), jnp.float32),
          pltpu.VMEM((b_per_w, E), jnp.float32),
          pltpu.SemaphoreType.DMA,
      ],
  )
  def pool(table_hbm, idx_hbm, out_hbm, idx_v, rows_v, avg_v, sem):
    wid = lax.axis_index("s") * info.num_cores + lax.axis_index("c")
    base = wid * b_per_w
    pltpu.sync_copy(idx_hbm.at[pl.ds(base * CTX, n_idx)], idx_v)
    copies = []
    for j in range(n_chunks):
      copies.append(
          pltpu.async_copy(
              table_hbm.at[idx_v.at[pl.ds(j * _IDX_CHUNK, _IDX_CHUNK)]],
              rows_v.at[pl.ds(j * _IDX_CHUNK, _IDX_CHUNK)],
              sem,
          )
      )
    for c in copies:
      c.wait()
    scale = jnp.float32(1.0 / CTX)

    def row_body(r, carry):
      rbase = r * CTX
      for e in range(E // _LANES):
        sl = pl.ds(e * _LANES, _LANES)
        acc = rows_v[rbase, sl]
        for c in range(1, CTX):
          acc = acc + rows_v[rbase + c, sl]
        avg_v[r, sl] = acc * scale
      return carry

    lax.fori_loop(0, b_per_w, row_body, 0)
    pltpu.sync_copy(avg_v, out_hbm.at[pl.ds(base, b_per_w)])

  return pool


@functools.lru_cache(maxsize=None)
def _make_matmul_t(B, E, V, BN=4096):
  grid = pl.cdiv(V, BN)

  def mm(avg_ref, w_ref, b_ref, out_ref):
    prod = lax.dot_general(
        w_ref[...], avg_ref[...],
        dimension_numbers=(((0,), (1,)), ((), ())),
        preferred_element_type=jnp.float32,
    )
    out_ref[...] = prod + b_ref[...]

  return pl.pallas_call(
      mm,
      grid=(grid,),
      in_specs=[
          pl.BlockSpec((B, E), lambda j: (0, 0)),
          pl.BlockSpec((E, BN), lambda j: (0, j)),
          pl.BlockSpec((BN, 1), lambda j: (j, 0)),
      ],
      out_specs=pl.BlockSpec((BN, B), lambda j: (j, 0)),
      out_shape=jax.ShapeDtypeStruct((V, B), jnp.float32),
  )


def kernel(x, emb_table, W, b):
  B, CTX = x.shape
  V, E = emb_table.shape
  avg = _make_pool(V, E, B, CTX)(emb_table, x.reshape(B * CTX))
  out_t = _make_matmul_t(B, E, V)(avg, W, b.reshape(V, 1))
  return out_t.T
